# baseline (speedup 1.0000x reference)
import functools

import jax
import jax.numpy as jnp
from jax import lax
from jax.experimental import pallas as pl
from jax.experimental.pallas import tpu as pltpu
from jax.experimental.pallas import tpu_sc as plsc

NUM_CORES = 2
NUM_SUBCORES = 16
LANES = 16
NUM_WORKERS = NUM_CORES * NUM_SUBCORES

ROWS = 16384
COLS = 200
ROWS_PER_WORKER = ROWS // NUM_WORKERS
NCHUNKS = 4
CHUNK_ROWS = ROWS_PER_WORKER // NCHUNKS
CHUNK_WORDS = CHUNK_ROWS * COLS
BLOCKS_PER_CHUNK = CHUNK_ROWS // LANES


@functools.partial(
    pl.kernel,
    out_type=(
        jax.ShapeDtypeStruct((ROWS * COLS,), jnp.int32),
        jax.ShapeDtypeStruct((ROWS,), jnp.float32),
    ),
    mesh=plsc.VectorSubcoreMesh(core_axis_name="c", subcore_axis_name="s"),
    compiler_params=pltpu.CompilerParams(needs_layout_passes=False),
    scratch_types=[
        pltpu.VMEM((CHUNK_WORDS,), jnp.int32),
        pltpu.VMEM((CHUNK_WORDS,), jnp.int32),
        pltpu.VMEM((ROWS_PER_WORKER,), jnp.float32),
        pltpu.VMEM((ROWS_PER_WORKER,), jnp.float32),
        pltpu.VMEM((LANES,), jnp.float32),
        pltpu.SemaphoreType.DMA,
        pltpu.SemaphoreType.DMA,
        pltpu.SemaphoreType.DMA,
        pltpu.SemaphoreType.DMA,
    ],
)
def _sae_add(species_hbm, energies_hbm, table_hbm, species_out_hbm, out_hbm,
             sp0, sp1, en_v, out_v, tab_v,
             sem_in0, sem_in1, sem_out0, sem_out1):
    wid = lax.axis_index("s") * NUM_CORES + lax.axis_index("c")
    base = wid * ROWS_PER_WORKER

    pltpu.sync_copy(table_hbm, tab_v)
    pltpu.sync_copy(energies_hbm.at[pl.ds(base, ROWS_PER_WORKER)], en_v)

    bufs = (sp0, sp1)
    in_sems = (sem_in0, sem_in1)
    out_sems = (sem_out0, sem_out1)

    def chunk_slice(g):
        return pl.ds(base * COLS + g * CHUNK_WORDS, CHUNK_WORDS)

    lane = jnp.arange(LANES, dtype=jnp.int32)
    ins = {0: pltpu.async_copy(species_hbm.at[chunk_slice(0)], bufs[0],
                               in_sems[0])}
    outs = {}
    for g in range(NCHUNKS):
        buf = bufs[g % 2]
        ins[g].wait()
        if g + 1 < NCHUNKS:
            if g - 1 >= 0:
                outs[g - 1].wait()
            nb = (g + 1) % 2
            ins[g + 1] = pltpu.async_copy(
                species_hbm.at[chunk_slice(g + 1)], bufs[nb], in_sems[nb])
        outs[g] = pltpu.async_copy(buf, species_out_hbm.at[chunk_slice(g)],
                                   out_sems[g % 2])

        for b in range(BLOCKS_PER_CHUNK):
            rowoff = (lane + b * LANES) * COLS

            def body(j, carry):
                acc, colv = carry
                sv = plsc.load_gather(buf, [rowoff + colv])
                tv = plsc.load_gather(tab_v, [sv & 15])
                return acc + tv, colv + 1

            acc, _ = lax.fori_loop(
                0, COLS, body,
                (jnp.zeros((LANES,), jnp.float32),
                 jnp.zeros((LANES,), jnp.int32)),
                unroll=8,
            )
            r0 = g * CHUNK_ROWS + b * LANES
            out_v[pl.ds(r0, LANES)] = acc + en_v[pl.ds(r0, LANES)]

    outs[NCHUNKS - 2].wait()
    outs[NCHUNKS - 1].wait()
    pltpu.sync_copy(out_v, out_hbm.at[pl.ds(base, ROWS_PER_WORKER)])


def kernel(species, energies, self_energies):
    table16 = jnp.pad(self_energies.astype(jnp.float32), (0, 16 - 10))
    species_flat = species.astype(jnp.int32).reshape(-1)
    species_out, new_energies = _sae_add(species_flat, energies, table16)
    return (species_out.reshape(species.shape).astype(species.dtype),
            new_energies)


# speedup vs baseline: 1.2494x; 1.2494x over previous
import functools

import jax
import jax.numpy as jnp
from jax import lax
from jax.experimental import pallas as pl
from jax.experimental.pallas import tpu as pltpu
from jax.experimental.pallas import tpu_sc as plsc

NUM_CORES = 2
NUM_SUBCORES = 16
LANES = 16
NUM_WORKERS = NUM_CORES * NUM_SUBCORES

ROWS = 16384
COLS = 200
ROWS_PER_WORKER = ROWS // NUM_WORKERS
NCHUNKS = 4
CHUNK_ROWS = ROWS_PER_WORKER // NCHUNKS
BLOCKS_PER_CHUNK = CHUNK_ROWS // LANES


@functools.partial(
    pl.kernel,
    out_type=jax.ShapeDtypeStruct((ROWS,), jnp.float32),
    mesh=plsc.VectorSubcoreMesh(core_axis_name="c", subcore_axis_name="s"),
    compiler_params=pltpu.CompilerParams(needs_layout_passes=False),
    scratch_types=[
        pltpu.VMEM((CHUNK_ROWS, COLS), jnp.int32),
        pltpu.VMEM((CHUNK_ROWS, COLS), jnp.int32),
        pltpu.VMEM((ROWS_PER_WORKER,), jnp.float32),
        pltpu.VMEM((ROWS_PER_WORKER,), jnp.float32),
        pltpu.VMEM((LANES,), jnp.float32),
        pltpu.SemaphoreType.DMA,
        pltpu.SemaphoreType.DMA,
    ],
)
def _sae_add(species_hbm, energies_hbm, table_hbm, out_hbm,
             sp0, sp1, en_v, out_v, tab_v, sem0, sem1):
    wid = lax.axis_index("s") * NUM_CORES + lax.axis_index("c")
    base = wid * ROWS_PER_WORKER

    pltpu.sync_copy(table_hbm, tab_v)
    pltpu.sync_copy(energies_hbm.at[pl.ds(base, ROWS_PER_WORKER)], en_v)

    bufs = (sp0, sp1)
    sems = (sem0, sem1)

    def start_in(g):
        return pltpu.async_copy(
            species_hbm.at[pl.ds(base + g * CHUNK_ROWS, CHUNK_ROWS)],
            bufs[g % 2], sems[g % 2])

    lane = jnp.arange(LANES, dtype=jnp.int32)
    ins = {0: start_in(0)}
    for g in range(NCHUNKS):
        buf = bufs[g % 2]
        ins[g].wait()
        if g + 1 < NCHUNKS:
            ins[g + 1] = start_in(g + 1)

        for b in range(BLOCKS_PER_CHUNK):
            rows = lane + b * LANES

            def body(j, carry):
                acc, colv = carry
                sv = plsc.load_gather(buf, [rows, colv])
                tv = plsc.load_gather(tab_v, [sv & 15])
                return acc + tv, colv + 1

            acc, _ = lax.fori_loop(
                0, COLS, body,
                (jnp.zeros((LANES,), jnp.float32),
                 jnp.zeros((LANES,), jnp.int32)),
                unroll=8,
            )
            r0 = g * CHUNK_ROWS + b * LANES
            out_v[pl.ds(r0, LANES)] = acc + en_v[pl.ds(r0, LANES)]

    pltpu.sync_copy(out_v, out_hbm.at[pl.ds(base, ROWS_PER_WORKER)])


def kernel(species, energies, self_energies):
    table16 = jnp.pad(self_energies.astype(jnp.float32), (0, 16 - 10))
    new_energies = _sae_add(species.astype(jnp.int32), energies, table16)
    return (species, new_energies)


# speedup vs baseline: 1.3564x; 1.0857x over previous
import functools

import jax
import jax.numpy as jnp
from jax import lax
from jax.experimental import pallas as pl
from jax.experimental.pallas import tpu as pltpu
from jax.experimental.pallas import tpu_sc as plsc

NUM_CORES = 2
NUM_SUBCORES = 16
LANES = 16
NUM_WORKERS = NUM_CORES * NUM_SUBCORES

ROWS = 16384
COLS = 200
ROWS_PER_WORKER = ROWS // NUM_WORKERS
NCHUNKS = 4
CHUNK_ROWS = ROWS_PER_WORKER // NCHUNKS
BLOCKS_PER_CHUNK = CHUNK_ROWS // LANES


@functools.partial(
    pl.kernel,
    out_type=jax.ShapeDtypeStruct((ROWS,), jnp.float32),
    mesh=plsc.VectorSubcoreMesh(core_axis_name="c", subcore_axis_name="s"),
    compiler_params=pltpu.CompilerParams(
        needs_layout_passes=False, use_tc_tiling_on_sc=False),
    scratch_types=[
        pltpu.VMEM((CHUNK_ROWS, COLS), jnp.int32),
        pltpu.VMEM((CHUNK_ROWS, COLS), jnp.int32),
        pltpu.VMEM((ROWS_PER_WORKER,), jnp.float32),
        pltpu.VMEM((ROWS_PER_WORKER,), jnp.float32),
        pltpu.VMEM((LANES,), jnp.float32),
        pltpu.SemaphoreType.DMA,
        pltpu.SemaphoreType.DMA,
    ],
)
def _sae_add(species_hbm, energies_hbm, table_hbm, out_hbm,
             sp0, sp1, en_v, out_v, tab_v, sem0, sem1):
    wid = lax.axis_index("s") * NUM_CORES + lax.axis_index("c")
    base = wid * ROWS_PER_WORKER

    pltpu.sync_copy(table_hbm, tab_v)
    pltpu.sync_copy(energies_hbm.at[pl.ds(base, ROWS_PER_WORKER)], en_v)

    bufs = (sp0, sp1)
    sems = (sem0, sem1)

    def start_in(g):
        return pltpu.async_copy(
            species_hbm.at[pl.ds(base + g * CHUNK_ROWS, CHUNK_ROWS)],
            bufs[g % 2], sems[g % 2])

    lane = jnp.arange(LANES, dtype=jnp.int32)
    ins = {0: start_in(0)}
    for g in range(NCHUNKS):
        buf = bufs[g % 2]
        ins[g].wait()
        if g + 1 < NCHUNKS:
            ins[g + 1] = start_in(g + 1)

        for b in range(BLOCKS_PER_CHUNK):
            rows = lane + b * LANES

            def body(j, carry):
                acc, colv = carry
                sv = plsc.load_gather(buf, [rows, colv])
                tv = plsc.load_gather(tab_v, [sv & 15])
                return acc + tv, colv + 1

            acc, _ = lax.fori_loop(
                0, COLS, body,
                (jnp.zeros((LANES,), jnp.float32),
                 jnp.zeros((LANES,), jnp.int32)),
                unroll=8,
            )
            r0 = g * CHUNK_ROWS + b * LANES
            out_v[pl.ds(r0, LANES)] = acc + en_v[pl.ds(r0, LANES)]

    pltpu.sync_copy(out_v, out_hbm.at[pl.ds(base, ROWS_PER_WORKER)])


def kernel(species, energies, self_energies):
    table16 = jnp.pad(self_energies.astype(jnp.float32), (0, 16 - 10))
    new_energies = _sae_add(species.astype(jnp.int32), energies, table16)
    return (species, new_energies)


# speedup vs baseline: 1.9011x; 1.4016x over previous
import functools

import jax
import jax.numpy as jnp
from jax import lax
from jax.experimental import pallas as pl
from jax.experimental.pallas import tpu as pltpu
from jax.experimental.pallas import tpu_sc as plsc

NUM_CORES = 2
NUM_SUBCORES = 16
LANES = 16
NUM_WORKERS = NUM_CORES * NUM_SUBCORES

ROWS = 16384
COLS = 200
ROWS_PER_WORKER = ROWS // NUM_WORKERS
NCHUNKS = 4
CHUNK_ROWS = ROWS_PER_WORKER // NCHUNKS
GROUPS_PER_CHUNK = CHUNK_ROWS // LANES
FULL_COL_CHUNKS = COLS // LANES
TAIL_START = COLS - LANES


@functools.partial(
    pl.kernel,
    out_type=(
        jax.ShapeDtypeStruct((ROWS, COLS), jnp.int32),
        jax.ShapeDtypeStruct((ROWS,), jnp.float32),
    ),
    mesh=plsc.VectorSubcoreMesh(core_axis_name="c", subcore_axis_name="s"),
    scratch_types=[
        pltpu.VMEM((CHUNK_ROWS, COLS), jnp.int32),
        pltpu.VMEM((CHUNK_ROWS, COLS), jnp.int32),
        pltpu.VMEM((ROWS_PER_WORKER,), jnp.float32),
        pltpu.VMEM((ROWS_PER_WORKER,), jnp.float32),
        pltpu.VMEM((LANES,), jnp.float32),
        pltpu.SemaphoreType.DMA,
        pltpu.SemaphoreType.DMA,
        pltpu.SemaphoreType.DMA,
        pltpu.SemaphoreType.DMA,
    ],
)
def _sae_add(species_hbm, energies_hbm, table_hbm, species_out_hbm, out_hbm,
             sp0, sp1, en_v, out_v, tab_v,
             sem_in0, sem_in1, sem_out0, sem_out1):
    wid = lax.axis_index("s") * NUM_CORES + lax.axis_index("c")
    base = wid * ROWS_PER_WORKER

    pltpu.sync_copy(table_hbm, tab_v)
    pltpu.sync_copy(energies_hbm.at[pl.ds(base, ROWS_PER_WORKER)], en_v)

    tab = tab_v[...]
    lane = jnp.arange(LANES, dtype=jnp.int32)
    tail_keep = lane >= (LANES - (COLS - FULL_COL_CHUNKS * LANES))
    butterfly = [lane ^ h for h in (8, 4, 2, 1)]

    bufs = (sp0, sp1)
    in_sems = (sem_in0, sem_in1)
    out_sems = (sem_out0, sem_out1)

    def row_slice(g):
        return pl.ds(base + g * CHUNK_ROWS, CHUNK_ROWS)

    def start_in(g):
        return pltpu.async_copy(
            species_hbm.at[row_slice(g)], bufs[g % 2], in_sems[g % 2])

    ins = {0: start_in(0)}
    outs = {}
    for g in range(NCHUNKS):
        buf = bufs[g % 2]
        ins[g].wait()
        if g + 1 < NCHUNKS:
            if g - 1 >= 0:
                outs[g - 1].wait()
            ins[g + 1] = start_in(g + 1)
        outs[g] = pltpu.async_copy(
            buf, species_out_hbm.at[row_slice(g)], out_sems[g % 2])

        def row_sum(r):
            acc = jnp.zeros((LANES,), jnp.float32)
            for ci in range(FULL_COL_CHUNKS):
                sv = buf[r, pl.ds(ci * LANES, LANES)]
                acc = acc + jnp.take_along_axis(tab, sv & 15, axis=0)
            svt = buf[r, pl.ds(TAIL_START, LANES)]
            tvt = jnp.take_along_axis(tab, svt & 15, axis=0)
            acc = acc + jnp.where(tail_keep, tvt, 0.0)
            for perm in butterfly:
                acc = acc + jnp.take_along_axis(acc, perm, axis=0)
            return acc

        def group_body(grp, _):
            def lane_body(k, res):
                s = row_sum(grp * LANES + k)
                return jnp.where(lane == k, s, res)
            res = lax.fori_loop(
                0, LANES, lane_body, jnp.zeros((LANES,), jnp.float32))
            row0 = g * CHUNK_ROWS + grp * LANES
            out_v[pl.ds(row0, LANES)] = res + en_v[pl.ds(row0, LANES)]
            return 0

        lax.fori_loop(0, GROUPS_PER_CHUNK, group_body, 0)

    outs[NCHUNKS - 2].wait()
    outs[NCHUNKS - 1].wait()
    pltpu.sync_copy(out_v, out_hbm.at[pl.ds(base, ROWS_PER_WORKER)])


def kernel(species, energies, self_energies):
    table16 = jnp.pad(self_energies.astype(jnp.float32), (0, 16 - 10))
    species_out, new_energies = _sae_add(species, energies, table16)
    return (species_out, new_energies)


# speedup vs baseline: 2.1616x; 1.1371x over previous
import functools

import jax
import jax.numpy as jnp
from jax import lax
from jax.experimental import pallas as pl
from jax.experimental.pallas import tpu as pltpu
from jax.experimental.pallas import tpu_sc as plsc

NUM_CORES = 2
NUM_SUBCORES = 16
LANES = 16
NUM_WORKERS = NUM_CORES * NUM_SUBCORES

ROWS = 16384
COLS = 200
ROWS_PER_WORKER = ROWS // NUM_WORKERS
NCHUNKS = 4
CHUNK_ROWS = ROWS_PER_WORKER // NCHUNKS
GROUPS_PER_CHUNK = CHUNK_ROWS // LANES
FULL_COL_CHUNKS = COLS // LANES
TAIL_START = COLS - LANES


@functools.partial(
    pl.kernel,
    out_type=jax.ShapeDtypeStruct((ROWS,), jnp.float32),
    mesh=plsc.VectorSubcoreMesh(core_axis_name="c", subcore_axis_name="s"),
    compiler_params=pltpu.CompilerParams(skip_device_barrier=True),
    scratch_types=[
        pltpu.VMEM((CHUNK_ROWS, COLS), jnp.int32),
        pltpu.VMEM((CHUNK_ROWS, COLS), jnp.int32),
        pltpu.VMEM((ROWS_PER_WORKER,), jnp.float32),
        pltpu.VMEM((ROWS_PER_WORKER,), jnp.float32),
        pltpu.VMEM((LANES,), jnp.float32),
        pltpu.SemaphoreType.DMA,
        pltpu.SemaphoreType.DMA,
    ],
)
def _sae_add(species_hbm, energies_hbm, table_hbm, out_hbm,
             sp0, sp1, en_v, out_v, tab_v, sem0, sem1):
    wid = lax.axis_index("s") * NUM_CORES + lax.axis_index("c")
    base = wid * ROWS_PER_WORKER

    pltpu.sync_copy(table_hbm, tab_v)
    pltpu.sync_copy(energies_hbm.at[pl.ds(base, ROWS_PER_WORKER)], en_v)

    tab = tab_v[...]
    lane = jnp.arange(LANES, dtype=jnp.int32)
    tail_keep = lane >= (LANES - (COLS - FULL_COL_CHUNKS * LANES))
    butterfly = [lane ^ h for h in (8, 4, 2, 1)]

    bufs = (sp0, sp1)
    sems = (sem0, sem1)

    def start_in(g):
        return pltpu.async_copy(
            species_hbm.at[pl.ds(base + g * CHUNK_ROWS, CHUNK_ROWS)],
            bufs[g % 2], sems[g % 2])

    ins = {0: start_in(0)}
    for g in range(NCHUNKS):
        buf = bufs[g % 2]
        ins[g].wait()
        if g + 1 < NCHUNKS:
            ins[g + 1] = start_in(g + 1)

        def row_sum(r):
            acc = jnp.zeros((LANES,), jnp.float32)
            for ci in range(FULL_COL_CHUNKS):
                sv = buf[r, pl.ds(ci * LANES, LANES)]
                acc = acc + jnp.take_along_axis(tab, sv & 15, axis=0)
            svt = buf[r, pl.ds(TAIL_START, LANES)]
            tvt = jnp.take_along_axis(tab, svt & 15, axis=0)
            acc = acc + jnp.where(tail_keep, tvt, 0.0)
            for perm in butterfly:
                acc = acc + jnp.take_along_axis(acc, perm, axis=0)
            return acc

        def group_body(grp, _):
            def lane_body(k, res):
                s = row_sum(grp * LANES + k)
                return jnp.where(lane == k, s, res)
            res = lax.fori_loop(
                0, LANES, lane_body, jnp.zeros((LANES,), jnp.float32))
            row0 = g * CHUNK_ROWS + grp * LANES
            out_v[pl.ds(row0, LANES)] = res + en_v[pl.ds(row0, LANES)]
            return 0

        lax.fori_loop(0, GROUPS_PER_CHUNK, group_body, 0)

    pltpu.sync_copy(out_v, out_hbm.at[pl.ds(base, ROWS_PER_WORKER)])


def kernel(species, energies, self_energies):
    table16 = jnp.pad(self_energies.astype(jnp.float32), (0, 16 - 10))
    new_energies = _sae_add(species, energies, table16)
    return (species, new_energies)


# speedup vs baseline: 2.1639x; 1.0011x over previous
import functools

import jax
import jax.numpy as jnp
from jax import lax
from jax.experimental import pallas as pl
from jax.experimental.pallas import tpu as pltpu
from jax.experimental.pallas import tpu_sc as plsc

NUM_CORES = 2
NUM_SUBCORES = 16
LANES = 16
NUM_WORKERS = NUM_CORES * NUM_SUBCORES

ROWS = 16384
COLS = 200
ROWS_PER_WORKER = ROWS // NUM_WORKERS
NCHUNKS = 4
CHUNK_ROWS = ROWS_PER_WORKER // NCHUNKS
GROUPS_PER_CHUNK = CHUNK_ROWS // LANES
FULL_COL_CHUNKS = COLS // LANES
TAIL_START = COLS - LANES


@functools.partial(
    pl.kernel,
    out_type=jax.ShapeDtypeStruct((ROWS,), jnp.float32),
    mesh=plsc.VectorSubcoreMesh(core_axis_name="c", subcore_axis_name="s"),
    compiler_params=pltpu.CompilerParams(skip_device_barrier=True),
    scratch_types=[
        pltpu.VMEM((CHUNK_ROWS, COLS), jnp.int32),
        pltpu.VMEM((CHUNK_ROWS, COLS), jnp.int32),
        pltpu.VMEM((ROWS_PER_WORKER,), jnp.float32),
        pltpu.VMEM((ROWS_PER_WORKER,), jnp.float32),
        pltpu.VMEM((LANES,), jnp.float32),
        pltpu.SemaphoreType.DMA,
        pltpu.SemaphoreType.DMA,
    ],
)
def _sae_add(species_hbm, energies_hbm, table_hbm, out_hbm,
             sp0, sp1, en_v, out_v, tab_v, sem0, sem1):
    wid = lax.axis_index("s") * NUM_CORES + lax.axis_index("c")
    base = wid * ROWS_PER_WORKER

    pltpu.sync_copy(table_hbm, tab_v)
    pltpu.sync_copy(energies_hbm.at[pl.ds(base, ROWS_PER_WORKER)], en_v)

    tab = tab_v[...]
    lane = jnp.arange(LANES, dtype=jnp.int32)
    tail_keep = lane >= (LANES - (COLS - FULL_COL_CHUNKS * LANES))
    butterfly = [lane ^ h for h in (8, 4, 2, 1)]

    bufs = (sp0, sp1)
    sems = (sem0, sem1)

    def start_in(g):
        return pltpu.async_copy(
            species_hbm.at[pl.ds(base + g * CHUNK_ROWS, CHUNK_ROWS)],
            bufs[g % 2], sems[g % 2])

    ins = {0: start_in(0)}
    for g in range(NCHUNKS):
        buf = bufs[g % 2]
        ins[g].wait()
        if g + 1 < NCHUNKS:
            ins[g + 1] = start_in(g + 1)

        def row_sum(r):
            def col_body(cb, acc):
                c0 = cb * (4 * LANES)
                for u in range(4):
                    sv = buf[r, pl.ds(c0 + u * LANES, LANES)]
                    acc = acc + jnp.take_along_axis(tab, sv & 15, axis=0)
                return acc
            acc = lax.fori_loop(
                0, FULL_COL_CHUNKS // 4, col_body,
                jnp.zeros((LANES,), jnp.float32))
            svt = buf[r, pl.ds(TAIL_START, LANES)]
            tvt = jnp.take_along_axis(tab, svt & 15, axis=0)
            acc = acc + jnp.where(tail_keep, tvt, 0.0)
            for perm in butterfly:
                acc = acc + jnp.take_along_axis(acc, perm, axis=0)
            return acc

        def group_body(grp, _):
            def lane_body(k, res):
                s = row_sum(grp * LANES + k)
                return jnp.where(lane == k, s, res)
            res = lax.fori_loop(
                0, LANES, lane_body, jnp.zeros((LANES,), jnp.float32))
            row0 = g * CHUNK_ROWS + grp * LANES
            out_v[pl.ds(row0, LANES)] = res + en_v[pl.ds(row0, LANES)]
            return 0

        lax.fori_loop(0, GROUPS_PER_CHUNK, group_body, 0)

    pltpu.sync_copy(out_v, out_hbm.at[pl.ds(base, ROWS_PER_WORKER)])


def kernel(species, energies, self_energies):
    table16 = jnp.pad(self_energies.astype(jnp.float32), (0, 16 - 10))
    new_energies = _sae_add(species, energies, table16)
    return (species, new_energies)
